# baseline (speedup 1.0000x reference)
import functools

import jax
import jax.numpy as jnp
from jax import lax
from jax.experimental import pallas as pl
from jax.experimental.pallas import tpu as pltpu
from jax.experimental.pallas import tpu_sc as plsc

B = 1024
L = 20
H = 512
QD = 512
VD = 2048

BB = 256
NB = B // BB
KEYS = 32

NC = 2
NS = 16
NW = NC * NS
ROWS_W = (B * L) // NW
CK = 128
NCHUNK = ROWS_W // CK

BF = jnp.bfloat16


def _prep_proj_kernel(v_ref, q_ref, cl_ref, g_ref, Wav_ref, Waq_ref,
                      bav_ref, baq_ref, bah_ref, Vfc_ref,
                      avq_ref, wfc_ref, idx_ref, nb_ref):
    f32 = jnp.float32
    b = pl.program_id(0)
    avq_ref[...] = (
        jnp.dot(v_ref[...].astype(BF), Wav_ref[...], preferred_element_type=f32)
        + jnp.dot(q_ref[...].astype(BF), Waq_ref[...], preferred_element_type=f32)
        + bav_ref[...] + baq_ref[...] + bah_ref[...])

    @pl.when(b == 0)
    def _():
        Vfc = Vfc_ref[...]
        ssq = jnp.sum(Vfc * Vfc)
        wfc_ref[...] = (Vfc * (lax.rsqrt(ssq) * g_ref[...])).astype(BF)

        cl = cl_ref[...]
        keys = lax.broadcasted_iota(jnp.int32, (B, KEYS), 1)
        onehot = (cl == keys).astype(f32)
        r_i = lax.broadcasted_iota(jnp.int32, (B, B), 0)
        c_j = lax.broadcasted_iota(jnp.int32, (B, B), 1)
        tri = (c_j <= r_i).astype(f32)
        cum = jnp.dot(tri, onehot, preferred_element_type=f32)
        counts = cum[B - 1:B, :]
        k_r = lax.broadcasted_iota(jnp.int32, (KEYS, KEYS), 0)
        k_c = lax.broadcasted_iota(jnp.int32, (KEYS, KEYS), 1)
        gt = (k_r > k_c).astype(f32)
        offs = jnp.dot(counts, gt, preferred_element_type=f32)
        pos = jnp.sum(onehot * (offs + cum), axis=1, keepdims=True) - 1.0
        nb_ref[...] = offs.astype(jnp.int32)
        c = lax.broadcasted_iota(jnp.int32, (B, L * 4), 1)
        idx_ref[...] = ((c & 3) * (L * B) + (c >> 2) * B
                        + pos.astype(jnp.int32))


def _prep_proj(v, q, cl2, g11, Wav_t, Waq_t, bav, baq, bah, Vfc_t):
    f32 = jnp.float32
    return pl.pallas_call(
        _prep_proj_kernel,
        grid=(NB,),
        in_specs=[
            pl.BlockSpec((BB, VD), lambda b: (b, 0)),
            pl.BlockSpec((BB, QD), lambda b: (b, 0)),
            pl.BlockSpec((B, 1), lambda b: (0, 0)),
            pl.BlockSpec((1, 1), lambda b: (0, 0)),
            pl.BlockSpec((VD, H), lambda b: (0, 0)),
            pl.BlockSpec((QD, H), lambda b: (0, 0)),
            pl.BlockSpec((1, H), lambda b: (0, 0)),
            pl.BlockSpec((1, H), lambda b: (0, 0)),
            pl.BlockSpec((1, H), lambda b: (0, 0)),
            pl.BlockSpec((H, H), lambda b: (0, 0)),
        ],
        out_specs=[
            pl.BlockSpec((BB, H), lambda b: (b, 0)),
            pl.BlockSpec((H, H), lambda b: (0, 0)),
            pl.BlockSpec((B, L * 4), lambda b: (0, 0)),
            pl.BlockSpec((1, KEYS), lambda b: (0, 0)),
        ],
        out_shape=[
            jax.ShapeDtypeStruct((B, H), f32),
            jax.ShapeDtypeStruct((H, H), BF),
            jax.ShapeDtypeStruct((B, L * 4), jnp.int32),
            jax.ShapeDtypeStruct((1, KEYS), jnp.int32),
        ],
    )(v, q, cl2, g11, Wav_t, Waq_t, bav, baq, bah, Vfc_t)


def _sc_permute(cap_lin, idx_flat):
    mesh = plsc.VectorSubcoreMesh(core_axis_name="c", subcore_axis_name="s")
    rows_w = (B * L * 4) // NW
    ck = 512
    nchunk = rows_w // ck

    @functools.partial(
        pl.kernel,
        out_type=jax.ShapeDtypeStruct((4 * L * B, 128), jnp.float32),
        mesh=mesh,
        scratch_types=[
            pltpu.VMEM((128,), jnp.int32),
            pltpu.VMEM((128,), jnp.int32),
            pltpu.VMEM((128,), jnp.int32),
            pltpu.VMEM((128,), jnp.int32),
            pltpu.VMEM((128, 128), jnp.float32),
            pltpu.VMEM((128, 128), jnp.float32),
            pltpu.VMEM((128, 128), jnp.float32),
            pltpu.VMEM((128, 128), jnp.float32),
            pltpu.SemaphoreType.DMA,
        ],
    )
    def k(cap_hbm, idx_hbm, out_hbm, i0, i1, i2, i3, b0, b1, b2, b3, sem):
        wid = lax.axis_index("s") * NC + lax.axis_index("c")
        base = wid * rows_w
        idxs = (i0, i1, i2, i3)
        bufs = (b0, b1, b2, b3)

        def body(c, carry):
            off = base + c * ck
            for kk in range(ck // 128):
                pltpu.sync_copy(idx_hbm.at[pl.ds(off + kk * 128, 128)],
                                idxs[kk])
                pltpu.sync_copy(cap_hbm.at[pl.ds(off + kk * 128, 128)],
                                bufs[kk])
            for kk in range(ck // 128):
                pltpu.async_copy(bufs[kk], out_hbm.at[idxs[kk]], sem).wait()
            return carry

        lax.fori_loop(0, nchunk, body, 0)

    return k(cap_lin, idx_flat)


def _rnn_kernel(nb_ref, x0_ref, x1_ref, x2_ref, x3_ref, avq_ref,
                Wihw_ref, Whhw_ref, Wihc_ref, Whhc_ref, Wah_ref, Wfc_ref,
                bihw_ref, bhhw_ref, bihc_ref, bhhc_ref, bfc_ref,
                out_ref, alp_ref, h1_ref, h2_ref,
                so_ref, sa_ref, sem_ref):
    f32 = jnp.float32
    b = pl.program_id(0)
    t = pl.program_id(1)
    base = b * BB
    nb_t = nb_ref[t]
    step = b * L + t
    slot = lax.rem(step, 2)

    @pl.when(step >= 2)
    def _():
        pltpu.make_async_copy(
            so_ref.at[slot], out_ref.at[pl.ds(base, BB), t], sem_ref.at[slot]
        ).wait()
        pltpu.make_async_copy(
            sa_ref.at[slot], alp_ref.at[pl.ds(base, BB), t], sem_ref.at[slot]
        ).wait()

    @pl.when(t == 0)
    def _():
        h1_ref[pl.ds(base, BB), :] = jnp.zeros((BB, H), f32)
        h2_ref[pl.ds(base, BB), :] = jnp.zeros((BB, H), f32)

    @pl.when(base < nb_t)
    def _():
        x = jnp.concatenate(
            [x0_ref[0], x1_ref[0], x2_ref[0], x3_ref[0]], axis=1)
        h1 = h1_ref[pl.ds(base, BB), :]
        h2 = h2_ref[pl.ds(base, BB), :]
        gi = jnp.dot(x.astype(BF), Wihw_ref[...],
                     preferred_element_type=f32) + bihw_ref[...]
        gh = jnp.dot(h1.astype(BF), Whhw_ref[...],
                     preferred_element_type=f32) + bhhw_ref[...]
        r = jax.nn.sigmoid(gi[:, :H] + gh[:, :H])
        z = jax.nn.sigmoid(gi[:, H:2 * H] + gh[:, H:2 * H])
        n = jnp.tanh(gi[:, 2 * H:] + r * gh[:, 2 * H:])
        h1n = (1.0 - z) * n + z * h1
        h1_ref[pl.ds(base, BB), :] = h1n

        att = jax.nn.sigmoid(
            jnp.dot(h1n.astype(BF), Wah_ref[...], preferred_element_type=f32)
            + avq_ref[...])

        xa = (att * x).astype(BF)
        gi2 = jnp.dot(xa, Wihc_ref[...], preferred_element_type=f32) + bihc_ref[...]
        gh2 = jnp.dot(h2.astype(BF), Whhc_ref[...],
                      preferred_element_type=f32) + bhhc_ref[...]
        r2 = jax.nn.sigmoid(gi2[:, :H] + gh2[:, :H])
        z2 = jax.nn.sigmoid(gi2[:, H:2 * H] + gh2[:, H:2 * H])
        n2 = jnp.tanh(gi2[:, 2 * H:] + r2 * gh2[:, 2 * H:])
        h2g = (1.0 - z2) * n2 + z2 * h2
        h2n = jnp.dot(h2g.astype(BF), Wfc_ref[...],
                      preferred_element_type=f32) + bfc_ref[...]
        h2_ref[pl.ds(base, BB), :] = h2n

        rows = base + lax.broadcasted_iota(jnp.int32, (BB, H), 0)
        m = rows < nb_t
        so_ref[slot] = jnp.where(m, h2n, 0.0)
        sa_ref[slot] = jnp.where(m, att, 0.0)

    @pl.when(base >= nb_t)
    def _():
        so_ref[slot] = jnp.zeros((BB, H), f32)
        sa_ref[slot] = jnp.zeros((BB, H), f32)

    pltpu.make_async_copy(
        so_ref.at[slot], out_ref.at[pl.ds(base, BB), t], sem_ref.at[slot]
    ).start()
    pltpu.make_async_copy(
        sa_ref.at[slot], alp_ref.at[pl.ds(base, BB), t], sem_ref.at[slot]
    ).start()

    @pl.when(step == NB * L - 1)
    def _():
        for s in (slot, 1 - slot):
            pltpu.make_async_copy(
                so_ref.at[s], out_ref.at[pl.ds(base, BB), t], sem_ref.at[s]
            ).wait()
            pltpu.make_async_copy(
                sa_ref.at[s], alp_ref.at[pl.ds(base, BB), t], sem_ref.at[s]
            ).wait()


def _rnn(nb, cap4, avq, Wihw_t, Whhw_t, Wihc_t, Whhc_t, Wah_t, Wfc_t,
         bihw, bhhw, bihc, bhhc, bfc):
    f32 = jnp.float32
    grid_spec = pltpu.PrefetchScalarGridSpec(
        num_scalar_prefetch=1,
        grid=(NB, L),
        in_specs=[
            pl.BlockSpec((1, BB, 128), lambda b, t, nb: (0, t * NB + b, 0)),
            pl.BlockSpec((1, BB, 128), lambda b, t, nb: (1, t * NB + b, 0)),
            pl.BlockSpec((1, BB, 128), lambda b, t, nb: (2, t * NB + b, 0)),
            pl.BlockSpec((1, BB, 128), lambda b, t, nb: (3, t * NB + b, 0)),
            pl.BlockSpec((BB, H), lambda b, t, nb: (b, 0)),
            pl.BlockSpec((QD, 3 * H), lambda b, t, nb: (0, 0)),
            pl.BlockSpec((H, 3 * H), lambda b, t, nb: (0, 0)),
            pl.BlockSpec((H, 3 * H), lambda b, t, nb: (0, 0)),
            pl.BlockSpec((H, 3 * H), lambda b, t, nb: (0, 0)),
            pl.BlockSpec((H, H), lambda b, t, nb: (0, 0)),
            pl.BlockSpec((H, H), lambda b, t, nb: (0, 0)),
            pl.BlockSpec((1, 3 * H), lambda b, t, nb: (0, 0)),
            pl.BlockSpec((1, 3 * H), lambda b, t, nb: (0, 0)),
            pl.BlockSpec((1, 3 * H), lambda b, t, nb: (0, 0)),
            pl.BlockSpec((1, 3 * H), lambda b, t, nb: (0, 0)),
            pl.BlockSpec((1, H), lambda b, t, nb: (0, 0)),
        ],
        out_specs=[
            pl.BlockSpec(memory_space=pl.ANY),
            pl.BlockSpec(memory_space=pl.ANY),
        ],
        scratch_shapes=[
            pltpu.VMEM((B, H), f32),
            pltpu.VMEM((B, H), f32),
            pltpu.VMEM((2, BB, H), f32),
            pltpu.VMEM((2, BB, H), f32),
            pltpu.SemaphoreType.DMA((2,)),
        ],
    )
    return pl.pallas_call(
        _rnn_kernel,
        grid_spec=grid_spec,
        out_shape=[
            jax.ShapeDtypeStruct((B, L, H), f32),
            jax.ShapeDtypeStruct((B, L, H), f32),
        ],
        compiler_params=pltpu.CompilerParams(
            dimension_semantics=("arbitrary", "arbitrary")),
    )(nb, cap4, cap4, cap4, cap4, avq, Wihw_t, Whhw_t, Wihc_t, Whhc_t,
      Wah_t, Wfc_t, bihw, bhhw, bihc, bhhc, bfc)


def kernel(v, q, caption, cap_len, W_ih_w, W_hh_w, b_ih_w, b_hh_w,
           W_ih_c, W_hh_c, b_ih_c, b_hh_c, W_ah, b_ah, W_av, b_av,
           W_aq, b_aq, V_fc, g_fc, b_fc):
    f32 = jnp.float32
    cl2 = cap_len.reshape(B, 1)
    g11 = jnp.asarray(g_fc, f32).reshape(1, 1)

    avq, wfc_t, idx2d, nb32 = _prep_proj(
        v, q, cl2, g11, W_av.T.astype(BF), W_aq.T.astype(BF),
        b_av.reshape(1, H), b_aq.reshape(1, H), b_ah.reshape(1, H), V_fc.T)
    nb = nb32[0, :L]

    cap4 = _sc_permute(caption.reshape(B * L * 4, 128),
                       idx2d.reshape(B * L * 4))
    cap4 = cap4.reshape(4, L * B, 128)

    out, alp = _rnn(
        nb, cap4, avq,
        W_ih_w.T.astype(BF), W_hh_w.T.astype(BF),
        W_ih_c.T.astype(BF), W_hh_c.T.astype(BF),
        W_ah.T.astype(BF), wfc_t,
        b_ih_w.reshape(1, 3 * H), b_hh_w.reshape(1, 3 * H),
        b_ih_c.reshape(1, 3 * H), b_hh_c.reshape(1, 3 * H),
        b_fc.reshape(1, H))
    return (out, alp)


# speedup vs baseline: 1.0484x; 1.0484x over previous
import functools

import jax
import jax.numpy as jnp
from jax import lax
from jax.experimental import pallas as pl
from jax.experimental.pallas import tpu as pltpu
from jax.experimental.pallas import tpu_sc as plsc

B = 1024
L = 20
H = 512
QD = 512
VD = 2048

BB = 256
NB = B // BB
KEYS = 32

NC = 2
NS = 16
NW = NC * NS
ROWS_W = (B * L) // NW
CK = 128
NCHUNK = ROWS_W // CK

BF = jnp.bfloat16


def _prep_proj_kernel(v_ref, q_ref, cl_ref, g_ref, Wav_ref, Waq_ref,
                      bav_ref, baq_ref, bah_ref, Vfc_ref,
                      avq_ref, wfc_ref, idx_ref, nb_ref):
    f32 = jnp.float32
    b = pl.program_id(0)
    avq_ref[...] = (
        jnp.dot(v_ref[...].astype(BF), Wav_ref[...], preferred_element_type=f32)
        + jnp.dot(q_ref[...].astype(BF), Waq_ref[...], preferred_element_type=f32)
        + bav_ref[...] + baq_ref[...] + bah_ref[...])

    @pl.when(b == 0)
    def _():
        Vfc = Vfc_ref[...]
        ssq = jnp.sum(Vfc * Vfc)
        wfc_ref[...] = (Vfc * (lax.rsqrt(ssq) * g_ref[...])).astype(BF)

        cl = cl_ref[...]
        keys = lax.broadcasted_iota(jnp.int32, (B, KEYS), 1)
        onehot = (cl == keys).astype(f32)
        r_i = lax.broadcasted_iota(jnp.int32, (B, B), 0)
        c_j = lax.broadcasted_iota(jnp.int32, (B, B), 1)
        tri = (c_j <= r_i).astype(f32)
        cum = jnp.dot(tri, onehot, preferred_element_type=f32)
        counts = cum[B - 1:B, :]
        k_r = lax.broadcasted_iota(jnp.int32, (KEYS, KEYS), 0)
        k_c = lax.broadcasted_iota(jnp.int32, (KEYS, KEYS), 1)
        gt = (k_r > k_c).astype(f32)
        offs = jnp.dot(counts, gt, preferred_element_type=f32)
        pos = jnp.sum(onehot * (offs + cum), axis=1, keepdims=True) - 1.0
        nb_ref[...] = offs.astype(jnp.int32)
        c = lax.broadcasted_iota(jnp.int32, (B, L * 4), 1)
        p = pos.astype(jnp.int32)
        idx_ref[...] = ((c >> 2) * (NB * 4 * BB) + (p // BB) * (4 * BB)
                        + (c & 3) * BB + (p % BB))


def _prep_proj(v, q, cl2, g11, Wav_t, Waq_t, bav, baq, bah, Vfc_t):
    f32 = jnp.float32
    return pl.pallas_call(
        _prep_proj_kernel,
        grid=(NB,),
        in_specs=[
            pl.BlockSpec((BB, VD), lambda b: (b, 0)),
            pl.BlockSpec((BB, QD), lambda b: (b, 0)),
            pl.BlockSpec((B, 1), lambda b: (0, 0)),
            pl.BlockSpec((1, 1), lambda b: (0, 0)),
            pl.BlockSpec((VD, H), lambda b: (0, 0)),
            pl.BlockSpec((QD, H), lambda b: (0, 0)),
            pl.BlockSpec((1, H), lambda b: (0, 0)),
            pl.BlockSpec((1, H), lambda b: (0, 0)),
            pl.BlockSpec((1, H), lambda b: (0, 0)),
            pl.BlockSpec((H, H), lambda b: (0, 0)),
        ],
        out_specs=[
            pl.BlockSpec((BB, H), lambda b: (b, 0)),
            pl.BlockSpec((H, H), lambda b: (0, 0)),
            pl.BlockSpec((B, L * 4), lambda b: (0, 0)),
            pl.BlockSpec((1, KEYS), lambda b: (0, 0)),
        ],
        out_shape=[
            jax.ShapeDtypeStruct((B, H), f32),
            jax.ShapeDtypeStruct((H, H), BF),
            jax.ShapeDtypeStruct((B, L * 4), jnp.int32),
            jax.ShapeDtypeStruct((1, KEYS), jnp.int32),
        ],
    )(v, q, cl2, g11, Wav_t, Waq_t, bav, baq, bah, Vfc_t)


def _sc_permute(cap_lin, idx_flat):
    mesh = plsc.VectorSubcoreMesh(core_axis_name="c", subcore_axis_name="s")
    rows_w = (B * L * 4) // NW
    ck = 512
    nchunk = rows_w // ck

    @functools.partial(
        pl.kernel,
        out_type=jax.ShapeDtypeStruct((4 * L * B, 128), jnp.float32),
        mesh=mesh,
        scratch_types=[
            pltpu.VMEM((128,), jnp.int32),
            pltpu.VMEM((128,), jnp.int32),
            pltpu.VMEM((128,), jnp.int32),
            pltpu.VMEM((128,), jnp.int32),
            pltpu.VMEM((128, 128), jnp.float32),
            pltpu.VMEM((128, 128), jnp.float32),
            pltpu.VMEM((128, 128), jnp.float32),
            pltpu.VMEM((128, 128), jnp.float32),
            pltpu.SemaphoreType.DMA,
        ],
    )
    def k(cap_hbm, idx_hbm, out_hbm, i0, i1, i2, i3, b0, b1, b2, b3, sem):
        wid = lax.axis_index("s") * NC + lax.axis_index("c")
        base = wid * rows_w
        idxs = (i0, i1, i2, i3)
        bufs = (b0, b1, b2, b3)

        def body(c, carry):
            off = base + c * ck
            for kk in range(ck // 128):
                pltpu.make_async_copy(idx_hbm.at[pl.ds(off + kk * 128, 128)],
                                      idxs[kk], sem).start()
                pltpu.make_async_copy(cap_hbm.at[pl.ds(off + kk * 128, 128)],
                                      bufs[kk], sem).start()
            for kk in range(ck // 128):
                pltpu.make_async_copy(idx_hbm.at[pl.ds(off + kk * 128, 128)],
                                      idxs[kk], sem).wait()
                pltpu.make_async_copy(cap_hbm.at[pl.ds(off + kk * 128, 128)],
                                      bufs[kk], sem).wait()
            for kk in range(ck // 128):
                pltpu.make_async_copy(bufs[kk], out_hbm.at[idxs[kk]],
                                      sem).start()
            for kk in range(ck // 128):
                pltpu.make_async_copy(bufs[kk], out_hbm.at[idxs[kk]],
                                      sem).wait()
            return carry

        lax.fori_loop(0, nchunk, body, 0)

    return k(cap_lin, idx_flat)


def _rnn_kernel(nb_ref, cap_ref, avq_ref,
                Wihw_ref, Whhw_ref, Wihc_ref, Whhc_ref, Wah_ref, Wfc_ref,
                bihw_ref, bhhw_ref, bihc_ref, bhhc_ref, bfc_ref,
                out_ref, alp_ref, h1_ref, h2_ref,
                so_ref, sa_ref, sem_ref):
    f32 = jnp.float32
    b = pl.program_id(0)
    t = pl.program_id(1)
    base = b * BB
    nb_t = nb_ref[t]
    step = b * L + t
    slot = lax.rem(step, 2)

    @pl.when(step >= 2)
    def _():
        pltpu.make_async_copy(
            so_ref.at[slot], out_ref.at[pl.ds(base, BB), t], sem_ref.at[slot]
        ).wait()
        pltpu.make_async_copy(
            sa_ref.at[slot], alp_ref.at[pl.ds(base, BB), t], sem_ref.at[slot]
        ).wait()

    @pl.when(t == 0)
    def _():
        h1_ref[pl.ds(base, BB), :] = jnp.zeros((BB, H), f32)
        h2_ref[pl.ds(base, BB), :] = jnp.zeros((BB, H), f32)

    @pl.when(base < nb_t)
    def _():
        x4 = cap_ref[...]
        x = jnp.concatenate(
            [x4[hj * BB:(hj + 1) * BB] for hj in range(4)], axis=1)
        h1 = h1_ref[pl.ds(base, BB), :]
        h2 = h2_ref[pl.ds(base, BB), :]
        gi = jnp.dot(x.astype(BF), Wihw_ref[...],
                     preferred_element_type=f32) + bihw_ref[...]
        gh = jnp.dot(h1.astype(BF), Whhw_ref[...],
                     preferred_element_type=f32) + bhhw_ref[...]
        r = jax.nn.sigmoid(gi[:, :H] + gh[:, :H])
        z = jax.nn.sigmoid(gi[:, H:2 * H] + gh[:, H:2 * H])
        n = jnp.tanh(gi[:, 2 * H:] + r * gh[:, 2 * H:])
        h1n = (1.0 - z) * n + z * h1
        h1_ref[pl.ds(base, BB), :] = h1n

        att = jax.nn.sigmoid(
            jnp.dot(h1n.astype(BF), Wah_ref[...], preferred_element_type=f32)
            + avq_ref[...])

        xa = (att * x).astype(BF)
        gi2 = jnp.dot(xa, Wihc_ref[...], preferred_element_type=f32) + bihc_ref[...]
        gh2 = jnp.dot(h2.astype(BF), Whhc_ref[...],
                      preferred_element_type=f32) + bhhc_ref[...]
        r2 = jax.nn.sigmoid(gi2[:, :H] + gh2[:, :H])
        z2 = jax.nn.sigmoid(gi2[:, H:2 * H] + gh2[:, H:2 * H])
        n2 = jnp.tanh(gi2[:, 2 * H:] + r2 * gh2[:, 2 * H:])
        h2g = (1.0 - z2) * n2 + z2 * h2
        h2n = jnp.dot(h2g.astype(BF), Wfc_ref[...],
                      preferred_element_type=f32) + bfc_ref[...]
        h2_ref[pl.ds(base, BB), :] = h2n

        rows = base + lax.broadcasted_iota(jnp.int32, (BB, H), 0)
        m = rows < nb_t
        so_ref[slot] = jnp.where(m, h2n, 0.0)
        sa_ref[slot] = jnp.where(m, att, 0.0)

    @pl.when(base >= nb_t)
    def _():
        so_ref[slot] = jnp.zeros((BB, H), f32)
        sa_ref[slot] = jnp.zeros((BB, H), f32)

    pltpu.make_async_copy(
        so_ref.at[slot], out_ref.at[pl.ds(base, BB), t], sem_ref.at[slot]
    ).start()
    pltpu.make_async_copy(
        sa_ref.at[slot], alp_ref.at[pl.ds(base, BB), t], sem_ref.at[slot]
    ).start()

    @pl.when(step == NB * L - 1)
    def _():
        for s in (slot, 1 - slot):
            pltpu.make_async_copy(
                so_ref.at[s], out_ref.at[pl.ds(base, BB), t], sem_ref.at[s]
            ).wait()
            pltpu.make_async_copy(
                sa_ref.at[s], alp_ref.at[pl.ds(base, BB), t], sem_ref.at[s]
            ).wait()


def _rnn(nb, cap4, avq, Wihw_t, Whhw_t, Wihc_t, Whhc_t, Wah_t, Wfc_t,
         bihw, bhhw, bihc, bhhc, bfc):
    f32 = jnp.float32
    grid_spec = pltpu.PrefetchScalarGridSpec(
        num_scalar_prefetch=1,
        grid=(NB, L),
        in_specs=[
            pl.BlockSpec((4 * BB, 128), lambda b, t, nb: (t * NB + b, 0)),
            pl.BlockSpec((BB, H), lambda b, t, nb: (b, 0)),
            pl.BlockSpec((QD, 3 * H), lambda b, t, nb: (0, 0)),
            pl.BlockSpec((H, 3 * H), lambda b, t, nb: (0, 0)),
            pl.BlockSpec((H, 3 * H), lambda b, t, nb: (0, 0)),
            pl.BlockSpec((H, 3 * H), lambda b, t, nb: (0, 0)),
            pl.BlockSpec((H, H), lambda b, t, nb: (0, 0)),
            pl.BlockSpec((H, H), lambda b, t, nb: (0, 0)),
            pl.BlockSpec((1, 3 * H), lambda b, t, nb: (0, 0)),
            pl.BlockSpec((1, 3 * H), lambda b, t, nb: (0, 0)),
            pl.BlockSpec((1, 3 * H), lambda b, t, nb: (0, 0)),
            pl.BlockSpec((1, 3 * H), lambda b, t, nb: (0, 0)),
            pl.BlockSpec((1, H), lambda b, t, nb: (0, 0)),
        ],
        out_specs=[
            pl.BlockSpec(memory_space=pl.ANY),
            pl.BlockSpec(memory_space=pl.ANY),
        ],
        scratch_shapes=[
            pltpu.VMEM((B, H), f32),
            pltpu.VMEM((B, H), f32),
            pltpu.VMEM((2, BB, H), f32),
            pltpu.VMEM((2, BB, H), f32),
            pltpu.SemaphoreType.DMA((2,)),
        ],
    )
    return pl.pallas_call(
        _rnn_kernel,
        grid_spec=grid_spec,
        out_shape=[
            jax.ShapeDtypeStruct((B, L, H), f32),
            jax.ShapeDtypeStruct((B, L, H), f32),
        ],
        compiler_params=pltpu.CompilerParams(
            dimension_semantics=("arbitrary", "arbitrary")),
    )(nb, cap4, avq, Wihw_t, Whhw_t, Wihc_t, Whhc_t,
      Wah_t, Wfc_t, bihw, bhhw, bihc, bhhc, bfc)


def kernel(v, q, caption, cap_len, W_ih_w, W_hh_w, b_ih_w, b_hh_w,
           W_ih_c, W_hh_c, b_ih_c, b_hh_c, W_ah, b_ah, W_av, b_av,
           W_aq, b_aq, V_fc, g_fc, b_fc):
    f32 = jnp.float32
    cl2 = cap_len.reshape(B, 1)
    g11 = jnp.asarray(g_fc, f32).reshape(1, 1)

    avq, wfc_t, idx2d, nb32 = _prep_proj(
        v, q, cl2, g11, W_av.T.astype(BF), W_aq.T.astype(BF),
        b_av.reshape(1, H), b_aq.reshape(1, H), b_ah.reshape(1, H), V_fc.T)
    nb = nb32[0, :L]

    cap4 = _sc_permute(caption.reshape(B * L * 4, 128),
                       idx2d.reshape(B * L * 4))

    out, alp = _rnn(
        nb, cap4, avq,
        W_ih_w.T.astype(BF), W_hh_w.T.astype(BF),
        W_ih_c.T.astype(BF), W_hh_c.T.astype(BF),
        W_ah.T.astype(BF), wfc_t,
        b_ih_w.reshape(1, 3 * H), b_hh_w.reshape(1, 3 * H),
        b_ih_c.reshape(1, 3 * H), b_hh_c.reshape(1, 3 * H),
        b_fc.reshape(1, H))
    return (out, alp)
